# baseline (speedup 1.0000x reference)
import jax
import jax.numpy as jnp
from jax.experimental import pallas as pl

N = 10000
E = 160000
SIGMA = 0.5
V = 1.0


def _q_body(z_ref, c_ref, q_ref):
    z = z_ref[...]
    c = c_ref[...]
    z2 = jnp.sum(z * z, axis=1, keepdims=True)
    c2 = jnp.sum(c * c, axis=1, keepdims=True).T
    qinv = 1.0 + (z2 + c2 - 2.0 * jnp.dot(z, c.T, preferred_element_type=jnp.float32)) / V
    q = 1.0 / qinv
    q = q ** ((V + 1.0) / 2.0)
    q_ref[...] = q / jnp.sum(q, axis=1, keepdims=True)


def _q_pallas(z, cluster):
    blk = 2000
    nc, nz = cluster.shape
    return pl.pallas_call(
        _q_body,
        grid=(N // blk,),
        in_specs=[
            pl.BlockSpec((blk, nz), lambda i: (i, 0)),
            pl.BlockSpec((nc, nz), lambda i: (0, 0)),
        ],
        out_specs=pl.BlockSpec((blk, nc), lambda i: (i, 0)),
        out_shape=jax.ShapeDtypeStruct((N, nc), jnp.float32),
    )(z, cluster)


def _gconv(h, W, src, dst, ninv_src, ninv_dst, act):
    h = (h * ninv_src[:, None]) @ W
    agg = jax.ops.segment_sum(h[src], dst, num_segments=N)
    out = agg * ninv_dst[:, None]
    return jax.nn.relu(out) if act else out


def kernel(x, edge_index, enc_W1, enc_b1, enc_W2, enc_b2, enc_W3, enc_b3, z_W, z_b,
           dec_W1, dec_b1, dec_W2, dec_b2, dec_W3, dec_b3, xbar_W, xbar_b,
           gW1, gW2, gW3, gW4, gW5, cluster):
    src = edge_index[0]
    dst = edge_index[1]
    ones = jnp.ones((E,), dtype=jnp.float32)
    deg_out = jax.ops.segment_sum(ones, src, num_segments=N)
    deg_in = jax.ops.segment_sum(ones, dst, num_segments=N)
    ninv_src = jnp.where(deg_out > 0, deg_out, 1.0) ** -0.5
    ninv_dst = jnp.where(deg_in > 0, deg_in, 1.0) ** -0.5
    tra1 = jax.nn.relu(x @ enc_W1 + enc_b1)
    tra2 = jax.nn.relu(tra1 @ enc_W2 + enc_b2)
    tra3 = jax.nn.relu(tra2 @ enc_W3 + enc_b3)
    z = tra3 @ z_W + z_b
    d1 = jax.nn.relu(z @ dec_W1 + dec_b1)
    d2 = jax.nn.relu(d1 @ dec_W2 + dec_b2)
    d3 = jax.nn.relu(d2 @ dec_W3 + dec_b3)
    x_bar = d3 @ xbar_W + xbar_b
    h = _gconv(x, gW1, src, dst, ninv_src, ninv_dst, True)
    h = _gconv((1.0 - SIGMA) * h + SIGMA * tra1, gW2, src, dst, ninv_src, ninv_dst, True)
    h = _gconv((1.0 - SIGMA) * h + SIGMA * tra2, gW3, src, dst, ninv_src, ninv_dst, True)
    h = _gconv((1.0 - SIGMA) * h + SIGMA * tra3, gW4, src, dst, ninv_src, ninv_dst, True)
    h = _gconv((1.0 - SIGMA) * h + SIGMA * z, gW5, src, dst, ninv_src, ninv_dst, False)
    predict = jax.nn.softmax(h, axis=1)
    q = _q_pallas(z, cluster)
    qd = jax.lax.stop_gradient(q)
    weight = qd ** 2 / jnp.sum(qd, axis=0)
    p = weight / jnp.sum(weight, axis=1, keepdims=True)
    return (x_bar, q, predict, p)


# speedup vs baseline: 3.5680x; 3.5680x over previous
import functools

import jax
import jax.numpy as jnp
from jax import lax
from jax.experimental import pallas as pl
from jax.experimental.pallas import tpu as pltpu
from jax.experimental.pallas import tpu_sc as plsc

N = 10000
E = 160000
SIGMA = 0.5
V = 1.0

B = 128
NB_A = 79
NB_B = 40
ACC_ROWS = 10112
TRASH = N
ZROWS = 632
WROWS = 624

_MESH = plsc.VectorSubcoreMesh(core_axis_name="c", subcore_axis_name="s")


def _make_agg_wide(nch):
    def body(*refs):
        ychs = refs[:nch]
        srcA, dstA, zeros_h, out, srci, dsti, gbuf, acc = refs[nch:]
        cid = lax.axis_index("c")
        sid = lax.axis_index("s")
        pltpu.sync_copy(srcA.at[sid], srci)
        pltpu.sync_copy(dstA.at[sid], dsti)
        for ch in range(nch):
            ych = ychs[ch]

            @pl.when(cid == (ch % 2))
            def _(ych=ych, ch=ch):
                pltpu.sync_copy(zeros_h, acc.at[pl.ds(ZROWS * sid, ZROWS)])
                plsc.subcore_barrier()

                @pl.loop(0, NB_A)
                def _(b):
                    pltpu.sync_copy(ych.at[srci.at[b]], gbuf)
                    pltpu.sync_copy(gbuf, acc.at[dsti.at[b]], add=True)

                plsc.subcore_barrier()
                pltpu.sync_copy(acc.at[pl.ds(WROWS * sid, WROWS)],
                                out.at[ch, pl.ds(WROWS * sid, WROWS)])

                @pl.when(sid == 0)
                def _(ch=ch):
                    pltpu.sync_copy(acc.at[pl.ds(16 * WROWS, N - 16 * WROWS)],
                                    out.at[ch, pl.ds(16 * WROWS, N - 16 * WROWS)])

                plsc.subcore_barrier()

    return pl.kernel(
        body,
        out_type=jax.ShapeDtypeStruct((nch, N, 128), jnp.float32),
        mesh=_MESH,
        scratch_types=[
            pltpu.VMEM((NB_A, B), jnp.int32),
            pltpu.VMEM((NB_A, B), jnp.int32),
            pltpu.VMEM((B, 128), jnp.float32),
            pltpu.VMEM_SHARED((ACC_ROWS, 128), jnp.float32),
        ],
    )


def _agg_narrow_body(ych, srcB, dstB, zeros_h, out, srci, dsti, gbuf, acc):
    cid = lax.axis_index("c")
    sid = lax.axis_index("s")
    w = cid * 16 + sid
    pltpu.sync_copy(srcB.at[w], srci)
    pltpu.sync_copy(dstB.at[w], dsti)
    pltpu.sync_copy(zeros_h, acc.at[pl.ds(ZROWS * sid, ZROWS)])
    plsc.subcore_barrier()

    @pl.loop(0, NB_B)
    def _(b):
        pltpu.sync_copy(ych.at[srci.at[b]], gbuf)
        pltpu.sync_copy(gbuf, acc.at[dsti.at[b]], add=True)

    plsc.subcore_barrier()
    pltpu.sync_copy(acc.at[pl.ds(WROWS * sid, WROWS)],
                    out.at[cid, pl.ds(WROWS * sid, WROWS)])

    @pl.when(sid == 0)
    def _():
        pltpu.sync_copy(acc.at[pl.ds(16 * WROWS, N - 16 * WROWS)],
                        out.at[cid, pl.ds(16 * WROWS, N - 16 * WROWS)])


_agg_narrow = pl.kernel(
    _agg_narrow_body,
    out_type=jax.ShapeDtypeStruct((2, N, 128), jnp.float32),
    mesh=_MESH,
    scratch_types=[
        pltpu.VMEM((NB_B, B), jnp.int32),
        pltpu.VMEM((NB_B, B), jnp.int32),
        pltpu.VMEM((B, 128), jnp.float32),
        pltpu.VMEM_SHARED((ACC_ROWS, 128), jnp.float32),
    ],
)


def _degrees_body(srcA_t, dstA, zeros_h, ones_h, out, idxv, obuf, acc):
    cid = lax.axis_index("c")
    sid = lax.axis_index("s")

    @pl.when(cid == 0)
    def _():
        pltpu.sync_copy(dstA.at[sid], idxv)

    @pl.when(cid == 1)
    def _():
        pltpu.sync_copy(srcA_t.at[sid], idxv)

    pltpu.sync_copy(ones_h, obuf)
    pltpu.sync_copy(zeros_h, acc.at[pl.ds(ZROWS * sid, ZROWS)])
    plsc.subcore_barrier()

    @pl.loop(0, NB_A)
    def _(b):
        pltpu.sync_copy(obuf, acc.at[idxv.at[b]], add=True)

    plsc.subcore_barrier()
    pltpu.sync_copy(acc.at[pl.ds(WROWS * sid, WROWS)],
                    out.at[cid, pl.ds(WROWS * sid, WROWS)])

    @pl.when(sid == 0)
    def _():
        pltpu.sync_copy(acc.at[pl.ds(16 * WROWS, N - 16 * WROWS)],
                        out.at[cid, pl.ds(16 * WROWS, N - 16 * WROWS)])


_degrees = pl.kernel(
    _degrees_body,
    out_type=jax.ShapeDtypeStruct((2, N, 128), jnp.float32),
    mesh=_MESH,
    scratch_types=[
        pltpu.VMEM((NB_A, B), jnp.int32),
        pltpu.VMEM((B, 128), jnp.float32),
        pltpu.VMEM_SHARED((ACC_ROWS, 128), jnp.float32),
    ],
)


def _edge_layout(idx, ways, nb, pad_val):
    per = E // ways
    cap = nb * B
    r = idx.reshape(ways, per)
    pad = jnp.full((ways, cap - per), pad_val, dtype=jnp.int32)
    return jnp.concatenate([r, pad], axis=1).reshape(ways, nb, B)


def _chunked(y, nch):
    w = y.shape[1]
    if w < nch * 128:
        y = jnp.pad(y, ((0, 0), (0, nch * 128 - w)))
    return [y[:, 128 * i:128 * (i + 1)] for i in range(nch)]


def _agg_wide(y, nch, srcA, dstA, zeros128):
    out = _make_agg_wide(nch)(*_chunked(y, nch), srcA, dstA, zeros128)
    return out.transpose(1, 0, 2).reshape(N, nch * 128)


def _agg16(y, srcB, dstB, zeros128):
    yp = jnp.pad(y, ((0, 0), (0, 128 - y.shape[1])))
    out = _agg_narrow(yp, srcB, dstB, zeros128)
    return out[0] + out[1]


def _q_body(z_ref, c_ref, q_ref):
    z = z_ref[...]
    c = c_ref[...]
    z2 = jnp.sum(z * z, axis=1, keepdims=True)
    c2 = jnp.sum(c * c, axis=1, keepdims=True).T
    qinv = 1.0 + (z2 + c2 - 2.0 * jnp.dot(z, c.T, preferred_element_type=jnp.float32)) / V
    q = 1.0 / qinv
    q = q ** ((V + 1.0) / 2.0)
    q_ref[...] = q / jnp.sum(q, axis=1, keepdims=True)


def _q_pallas(z, cluster):
    blk = 2000
    nc, nz = cluster.shape
    return pl.pallas_call(
        _q_body,
        grid=(N // blk,),
        in_specs=[
            pl.BlockSpec((blk, nz), lambda i: (i, 0)),
            pl.BlockSpec((nc, nz), lambda i: (0, 0)),
        ],
        out_specs=pl.BlockSpec((blk, nc), lambda i: (i, 0)),
        out_shape=jax.ShapeDtypeStruct((N, nc), jnp.float32),
    )(z, cluster)


def kernel(x, edge_index, enc_W1, enc_b1, enc_W2, enc_b2, enc_W3, enc_b3, z_W, z_b,
           dec_W1, dec_b1, dec_W2, dec_b2, dec_W3, dec_b3, xbar_W, xbar_b,
           gW1, gW2, gW3, gW4, gW5, cluster):
    src = edge_index[0]
    dst = edge_index[1]
    srcA = _edge_layout(src, 16, NB_A, 0)
    dstA = _edge_layout(dst, 16, NB_A, TRASH)
    srcB = _edge_layout(src, 32, NB_B, 0)
    dstB = _edge_layout(dst, 32, NB_B, TRASH)
    srcA_t = _edge_layout(src, 16, NB_A, TRASH)
    zeros128 = jnp.zeros((ZROWS, 128), jnp.float32)
    ones128 = jnp.ones((B, 128), jnp.float32)

    deg = _degrees(srcA_t, dstA, zeros128, ones128)
    deg_in = deg[0, :, 0]
    deg_out = deg[1, :, 0]
    ninv_src = jnp.where(deg_out > 0, deg_out, 1.0) ** -0.5
    ninv_dst = jnp.where(deg_in > 0, deg_in, 1.0) ** -0.5

    tra1 = jax.nn.relu(x @ enc_W1 + enc_b1)
    tra2 = jax.nn.relu(tra1 @ enc_W2 + enc_b2)
    tra3 = jax.nn.relu(tra2 @ enc_W3 + enc_b3)
    z = tra3 @ z_W + z_b
    d1 = jax.nn.relu(z @ dec_W1 + dec_b1)
    d2 = jax.nn.relu(d1 @ dec_W2 + dec_b2)
    d3 = jax.nn.relu(d2 @ dec_W3 + dec_b3)
    x_bar = d3 @ xbar_W + xbar_b

    y1 = x * ninv_src[:, None]
    agg1 = _agg_wide(y1, 2, srcA, dstA, zeros128)
    h = jax.nn.relu((agg1 * ninv_dst[:, None]) @ gW1)
    y2 = ((1.0 - SIGMA) * h + SIGMA * tra1) * ninv_src[:, None]
    agg2 = _agg_wide(y2, 4, srcA, dstA, zeros128)
    h = jax.nn.relu((agg2[:, :500] * ninv_dst[:, None]) @ gW2)
    t3 = (((1.0 - SIGMA) * h + SIGMA * tra2) * ninv_src[:, None]) @ gW3
    agg3 = _agg_wide(t3, 2, srcA, dstA, zeros128)
    h = jax.nn.relu(agg3[:, :200] * ninv_dst[:, None])
    t4 = (((1.0 - SIGMA) * h + SIGMA * tra3) * ninv_src[:, None]) @ gW4
    agg4 = _agg16(t4, srcB, dstB, zeros128)
    h = jax.nn.relu(agg4[:, :10] * ninv_dst[:, None])
    t5 = (((1.0 - SIGMA) * h + SIGMA * z) * ninv_src[:, None]) @ gW5
    agg5 = _agg16(t5, srcB, dstB, zeros128)
    h = agg5[:, :10] * ninv_dst[:, None]
    predict = jax.nn.softmax(h, axis=1)

    q = _q_pallas(z, cluster)
    qd = jax.lax.stop_gradient(q)
    weight = qd ** 2 / jnp.sum(qd, axis=0)
    p = weight / jnp.sum(weight, axis=1, keepdims=True)
    return (x_bar, q, predict, p)


# speedup vs baseline: 3.7423x; 1.0488x over previous
import functools

import jax
import jax.numpy as jnp
from jax import lax
from jax.experimental import pallas as pl
from jax.experimental.pallas import tpu as pltpu
from jax.experimental.pallas import tpu_sc as plsc

N = 10000
E = 160000
SIGMA = 0.5
V = 1.0

B = 128
NB_A = 79
NB_B = 40
ACC_ROWS = 10112
TRASH = N
ZROWS = 632
WROWS = 624

_MESH = plsc.VectorSubcoreMesh(core_axis_name="c", subcore_axis_name="s")


@functools.cache
def _make_agg_wide(nch):
    def body(*refs):
        ychs = refs[:nch]
        srcA, dstA, zeros_h, out, srci, dsti, gbuf, acc = refs[nch:]
        cid = lax.axis_index("c")
        sid = lax.axis_index("s")
        pltpu.sync_copy(srcA.at[sid], srci)
        pltpu.sync_copy(dstA.at[sid], dsti)
        for ch in range(nch):
            ych = ychs[ch]

            @pl.when(cid == (ch % 2))
            def _(ych=ych, ch=ch):
                pltpu.sync_copy(zeros_h, acc.at[pl.ds(ZROWS * sid, ZROWS)])
                plsc.subcore_barrier()

                @pl.loop(0, NB_A)
                def _(b):
                    pltpu.sync_copy(ych.at[srci.at[b]], gbuf)
                    pltpu.sync_copy(gbuf, acc.at[dsti.at[b]], add=True)

                plsc.subcore_barrier()
                pltpu.sync_copy(acc.at[pl.ds(WROWS * sid, WROWS)],
                                out.at[ch, pl.ds(WROWS * sid, WROWS)])

                @pl.when(sid == 0)
                def _(ch=ch):
                    pltpu.sync_copy(acc.at[pl.ds(16 * WROWS, N - 16 * WROWS)],
                                    out.at[ch, pl.ds(16 * WROWS, N - 16 * WROWS)])

                plsc.subcore_barrier()

    return pl.kernel(
        body,
        out_type=jax.ShapeDtypeStruct((nch, N, 128), jnp.float32),
        mesh=_MESH,
        scratch_types=[
            pltpu.VMEM((NB_A, B), jnp.int32),
            pltpu.VMEM((NB_A, B), jnp.int32),
            pltpu.VMEM((B, 128), jnp.float32),
            pltpu.VMEM_SHARED((ACC_ROWS, 128), jnp.float32),
        ],
    )


def _agg_narrow_body(ych, srcB, dstB, zeros_h, out, srci, dsti, gbuf, acc):
    cid = lax.axis_index("c")
    sid = lax.axis_index("s")
    w = cid * 16 + sid
    pltpu.sync_copy(srcB.at[w], srci)
    pltpu.sync_copy(dstB.at[w], dsti)
    pltpu.sync_copy(zeros_h, acc.at[pl.ds(ZROWS * sid, ZROWS)])
    plsc.subcore_barrier()

    @pl.loop(0, NB_B)
    def _(b):
        pltpu.sync_copy(ych.at[srci.at[b]], gbuf)
        pltpu.sync_copy(gbuf, acc.at[dsti.at[b]], add=True)

    plsc.subcore_barrier()
    pltpu.sync_copy(acc.at[pl.ds(WROWS * sid, WROWS)],
                    out.at[cid, pl.ds(WROWS * sid, WROWS)])

    @pl.when(sid == 0)
    def _():
        pltpu.sync_copy(acc.at[pl.ds(16 * WROWS, N - 16 * WROWS)],
                        out.at[cid, pl.ds(16 * WROWS, N - 16 * WROWS)])


_agg_narrow = pl.kernel(
    _agg_narrow_body,
    out_type=jax.ShapeDtypeStruct((2, N, 128), jnp.float32),
    mesh=_MESH,
    scratch_types=[
        pltpu.VMEM((NB_B, B), jnp.int32),
        pltpu.VMEM((NB_B, B), jnp.int32),
        pltpu.VMEM((B, 128), jnp.float32),
        pltpu.VMEM_SHARED((ACC_ROWS, 128), jnp.float32),
    ],
)


def _degrees_body(srcA_t, dstA, zeros_h, ones_h, out, idxv, obuf, acc):
    cid = lax.axis_index("c")
    sid = lax.axis_index("s")

    @pl.when(cid == 0)
    def _():
        pltpu.sync_copy(dstA.at[sid], idxv)

    @pl.when(cid == 1)
    def _():
        pltpu.sync_copy(srcA_t.at[sid], idxv)

    pltpu.sync_copy(ones_h, obuf)
    pltpu.sync_copy(zeros_h, acc.at[pl.ds(ZROWS * sid, ZROWS)])
    plsc.subcore_barrier()

    @pl.loop(0, NB_A)
    def _(b):
        pltpu.sync_copy(obuf, acc.at[idxv.at[b]], add=True)

    plsc.subcore_barrier()
    pltpu.sync_copy(acc.at[pl.ds(WROWS * sid, WROWS)],
                    out.at[cid, pl.ds(WROWS * sid, WROWS)])

    @pl.when(sid == 0)
    def _():
        pltpu.sync_copy(acc.at[pl.ds(16 * WROWS, N - 16 * WROWS)],
                        out.at[cid, pl.ds(16 * WROWS, N - 16 * WROWS)])


_degrees = pl.kernel(
    _degrees_body,
    out_type=jax.ShapeDtypeStruct((2, N, 128), jnp.float32),
    mesh=_MESH,
    scratch_types=[
        pltpu.VMEM((NB_A, B), jnp.int32),
        pltpu.VMEM((B, 128), jnp.float32),
        pltpu.VMEM_SHARED((ACC_ROWS, 128), jnp.float32),
    ],
)


def _edge_layout(idx, ways, nb, pad_val):
    per = E // ways
    cap = nb * B
    r = idx.reshape(ways, per)
    pad = jnp.full((ways, cap - per), pad_val, dtype=jnp.int32)
    return jnp.concatenate([r, pad], axis=1).reshape(ways, nb, B)


def _chunked(y, nch):
    w = y.shape[1]
    if w < nch * 128:
        y = jnp.pad(y, ((0, 0), (0, nch * 128 - w)))
    return [y[:, 128 * i:128 * (i + 1)] for i in range(nch)]


R = 1000
NBLK = N // R

_row = lambda i: (i, 0)
_bcast = lambda i: (0, 0)
_agg_spec = lambda nch: pl.BlockSpec((nch, R, 128), lambda i: (0, i, 0))


def _mm(a, w):
    return jnp.dot(a, w, preferred_element_type=jnp.float32)


def _relu(a):
    return jnp.maximum(a, 0.0)


def _cat_chunks(ref, nch, w):
    return jnp.concatenate([ref[c] for c in range(nch)], axis=1)[:, :w]


def _tcA_body(x_ref, deg_ref, eW1, eb1, eW2, eb2, eW3, eb3, zW, zb,
              dW1, db1, dW2, db2, dW3, db3, xW, xb, clus,
              nsrc_ref, ndst_ref, xs0_ref, xs1_ref, tra1_ref, tra2_ref,
              tra3_ref, z_ref, xbar_ref, q_ref, qcol_ref):
    x = x_ref[...]
    deg_in = deg_ref[0, :, 0:1]
    deg_out = deg_ref[1, :, 0:1]
    ninv_src = jax.lax.rsqrt(jnp.where(deg_out > 0, deg_out, 1.0))
    ninv_dst = jax.lax.rsqrt(jnp.where(deg_in > 0, deg_in, 1.0))
    nsrc_ref[...] = ninv_src
    ndst_ref[...] = ninv_dst
    xs = x * ninv_src
    xs0_ref[...] = xs[:, :128]
    xs1_ref[...] = xs[:, 128:]
    t1 = _relu(_mm(x, eW1[...]) + eb1[...])
    t2 = _relu(_mm(t1, eW2[...]) + eb2[...])
    t3 = _relu(_mm(t2, eW3[...]) + eb3[...])
    z = _mm(t3, zW[...]) + zb[...]
    tra1_ref[...] = t1
    tra2_ref[...] = t2
    tra3_ref[...] = t3
    z_ref[...] = z
    d1 = _relu(_mm(z, dW1[...]) + db1[...])
    d2 = _relu(_mm(d1, dW2[...]) + db2[...])
    d3 = _relu(_mm(d2, dW3[...]) + db3[...])
    xbar_ref[...] = _mm(d3, xW[...]) + xb[...]
    c = clus[...]
    z2 = jnp.sum(z * z, axis=1, keepdims=True)
    c2 = jnp.sum(c * c, axis=1, keepdims=True).T
    qinv = 1.0 + (z2 + c2 - 2.0 * _mm(z, c.T)) / V
    q = 1.0 / qinv
    q = q ** ((V + 1.0) / 2.0)
    q = q / jnp.sum(q, axis=1, keepdims=True)
    q_ref[...] = q
    qcol_ref[...] = jnp.sum(q, axis=0, keepdims=True)[None]


def _tcA(x, deg, ws):
    f32 = jnp.float32
    outs = [
        jax.ShapeDtypeStruct((N, 1), f32),
        jax.ShapeDtypeStruct((N, 1), f32),
        jax.ShapeDtypeStruct((N, 128), f32),
        jax.ShapeDtypeStruct((N, 128), f32),
        jax.ShapeDtypeStruct((N, 500), f32),
        jax.ShapeDtypeStruct((N, 500), f32),
        jax.ShapeDtypeStruct((N, 200), f32),
        jax.ShapeDtypeStruct((N, 10), f32),
        jax.ShapeDtypeStruct((N, 256), f32),
        jax.ShapeDtypeStruct((N, 10), f32),
        jax.ShapeDtypeStruct((NBLK, 1, 10), f32),
    ]
    in_specs = [pl.BlockSpec((R, 256), _row), _agg_spec(2)]
    in_specs += [pl.BlockSpec(w.shape, _bcast) for w in ws]
    out_specs = [pl.BlockSpec((R, s.shape[1]), _row) for s in outs[:-1]]
    out_specs.append(pl.BlockSpec((1, 1, 10), lambda i: (i, 0, 0)))
    return pl.pallas_call(
        _tcA_body, grid=(NBLK,), in_specs=in_specs, out_specs=out_specs,
        out_shape=outs,
    )(x, deg, *ws)


def _mix_scale(h, tra, ninv_src):
    return ((1.0 - SIGMA) * h + SIGMA * tra) * ninv_src


def _pad_cols(a, w):
    return jnp.concatenate(
        [a, jnp.zeros((a.shape[0], w - a.shape[1]), a.dtype)], axis=1)


def _tcB_body(agg_ref, nsrc_ref, ndst_ref, tra1_ref, gW1_ref,
              y0_ref, y1_ref, y2_ref, y3_ref):
    agg = _cat_chunks(agg_ref, 2, 256)
    h = _relu(_mm(agg * ndst_ref[...], gW1_ref[...]))
    y = _pad_cols(_mix_scale(h, tra1_ref[...], nsrc_ref[...]), 512)
    y0_ref[...] = y[:, 0:128]
    y1_ref[...] = y[:, 128:256]
    y2_ref[...] = y[:, 256:384]
    y3_ref[...] = y[:, 384:512]


def _tcB(agg1, nsrc, ndst, tra1, gW1):
    f32 = jnp.float32
    outs = [jax.ShapeDtypeStruct((N, 128), f32)] * 4
    return pl.pallas_call(
        _tcB_body, grid=(NBLK,),
        in_specs=[_agg_spec(2), pl.BlockSpec((R, 1), _row),
                  pl.BlockSpec((R, 1), _row), pl.BlockSpec((R, 500), _row),
                  pl.BlockSpec(gW1.shape, _bcast)],
        out_specs=[pl.BlockSpec((R, 128), _row)] * 4,
        out_shape=outs,
    )(agg1, nsrc, ndst, tra1, gW1)


def _tcC_body(agg_ref, nsrc_ref, ndst_ref, tra2_ref, gW2_ref, gW3_ref,
              t0_ref, t1_ref):
    agg = _cat_chunks(agg_ref, 4, 500)
    h = _relu(_mm(agg * ndst_ref[...], gW2_ref[...]))
    t = _mm(_mix_scale(h, tra2_ref[...], nsrc_ref[...]), gW3_ref[...])
    t = _pad_cols(t, 256)
    t0_ref[...] = t[:, 0:128]
    t1_ref[...] = t[:, 128:256]


def _tcC(agg2, nsrc, ndst, tra2, gW2, gW3):
    f32 = jnp.float32
    outs = [jax.ShapeDtypeStruct((N, 128), f32)] * 2
    return pl.pallas_call(
        _tcC_body, grid=(NBLK,),
        in_specs=[_agg_spec(4), pl.BlockSpec((R, 1), _row),
                  pl.BlockSpec((R, 1), _row), pl.BlockSpec((R, 500), _row),
                  pl.BlockSpec(gW2.shape, _bcast), pl.BlockSpec(gW3.shape, _bcast)],
        out_specs=[pl.BlockSpec((R, 128), _row)] * 2,
        out_shape=outs,
    )(agg2, nsrc, ndst, tra2, gW2, gW3)


def _tcD_body(agg_ref, nsrc_ref, ndst_ref, tra3_ref, gW4_ref, t_ref):
    agg = _cat_chunks(agg_ref, 2, 200)
    h = _relu(agg * ndst_ref[...])
    t = _mm(_mix_scale(h, tra3_ref[...], nsrc_ref[...]), gW4_ref[...])
    t_ref[...] = _pad_cols(t, 128)


def _tcD(agg3, nsrc, ndst, tra3, gW4):
    return pl.pallas_call(
        _tcD_body, grid=(NBLK,),
        in_specs=[_agg_spec(2), pl.BlockSpec((R, 1), _row),
                  pl.BlockSpec((R, 1), _row), pl.BlockSpec((R, 200), _row),
                  pl.BlockSpec(gW4.shape, _bcast)],
        out_specs=pl.BlockSpec((R, 128), _row),
        out_shape=jax.ShapeDtypeStruct((N, 128), jnp.float32),
    )(agg3, nsrc, ndst, tra3, gW4)


def _tcE_body(agg_ref, nsrc_ref, ndst_ref, z_ref, gW5_ref, t_ref):
    agg = (agg_ref[0] + agg_ref[1])[:, :10]
    h = _relu(agg * ndst_ref[...])
    t = _mm(_mix_scale(h, z_ref[...], nsrc_ref[...]), gW5_ref[...])
    t_ref[...] = _pad_cols(t, 128)


def _tcE(agg4, nsrc, ndst, z, gW5):
    return pl.pallas_call(
        _tcE_body, grid=(NBLK,),
        in_specs=[_agg_spec(2), pl.BlockSpec((R, 1), _row),
                  pl.BlockSpec((R, 1), _row), pl.BlockSpec((R, 10), _row),
                  pl.BlockSpec(gW5.shape, _bcast)],
        out_specs=pl.BlockSpec((R, 128), _row),
        out_shape=jax.ShapeDtypeStruct((N, 128), jnp.float32),
    )(agg4, nsrc, ndst, z, gW5)


def _tcF_body(agg_ref, ndst_ref, q_ref, qcol_ref, pred_ref, p_ref):
    agg = (agg_ref[0] + agg_ref[1])[:, :10]
    h = agg * ndst_ref[...]
    pred_ref[...] = jax.nn.softmax(h, axis=1)
    q = q_ref[...]
    qcol = jnp.sum(qcol_ref[...], axis=0)
    weight = (q * q) / qcol
    p_ref[...] = weight / jnp.sum(weight, axis=1, keepdims=True)


def _tcF(agg5, ndst, q, qcol):
    f32 = jnp.float32
    return pl.pallas_call(
        _tcF_body, grid=(NBLK,),
        in_specs=[_agg_spec(2), pl.BlockSpec((R, 1), _row),
                  pl.BlockSpec((R, 10), _row),
                  pl.BlockSpec((NBLK, 1, 10), lambda i: (0, 0, 0))],
        out_specs=[pl.BlockSpec((R, 10), _row)] * 2,
        out_shape=[jax.ShapeDtypeStruct((N, 10), f32)] * 2,
    )(agg5, ndst, q, qcol)


def kernel(x, edge_index, enc_W1, enc_b1, enc_W2, enc_b2, enc_W3, enc_b3, z_W, z_b,
           dec_W1, dec_b1, dec_W2, dec_b2, dec_W3, dec_b3, xbar_W, xbar_b,
           gW1, gW2, gW3, gW4, gW5, cluster):
    src = edge_index[0]
    dst = edge_index[1]
    srcA = _edge_layout(src, 16, NB_A, 0)
    dstA = _edge_layout(dst, 16, NB_A, TRASH)
    srcB = _edge_layout(src, 32, NB_B, 0)
    dstB = _edge_layout(dst, 32, NB_B, TRASH)
    srcA_t = _edge_layout(src, 16, NB_A, TRASH)
    zeros128 = jnp.zeros((ZROWS, 128), jnp.float32)
    ones128 = jnp.ones((B, 128), jnp.float32)

    deg = _degrees(srcA_t, dstA, zeros128, ones128)

    ws = [enc_W1, enc_b1.reshape(1, -1), enc_W2, enc_b2.reshape(1, -1),
          enc_W3, enc_b3.reshape(1, -1), z_W, z_b.reshape(1, -1),
          dec_W1, dec_b1.reshape(1, -1), dec_W2, dec_b2.reshape(1, -1),
          dec_W3, dec_b3.reshape(1, -1), xbar_W, xbar_b.reshape(1, -1),
          cluster]
    (nsrc, ndst, xs0, xs1, tra1, tra2, tra3, z, x_bar, q, qcol) = _tcA(x, deg, ws)

    agg1 = _make_agg_wide(2)(xs0, xs1, srcA, dstA, zeros128)
    y2c = _tcB(agg1, nsrc, ndst, tra1, gW1)
    agg2 = _make_agg_wide(4)(*y2c, srcA, dstA, zeros128)
    t3c = _tcC(agg2, nsrc, ndst, tra2, gW2, gW3)
    agg3 = _make_agg_wide(2)(*t3c, srcA, dstA, zeros128)
    t4 = _tcD(agg3, nsrc, ndst, tra3, gW4)
    agg4 = _agg_narrow(t4, srcB, dstB, zeros128)
    t5 = _tcE(agg4, nsrc, ndst, z, gW5)
    agg5 = _agg_narrow(t5, srcB, dstB, zeros128)
    predict, p = _tcF(agg5, ndst, q, qcol)
    return (x_bar, q, predict, p)
